# baseline (speedup 1.0000x reference)
import functools
import jax
import jax.numpy as jnp
from jax import lax
from jax.experimental import pallas as pl
from jax.experimental.pallas import tpu as pltpu, tpu_sc as plsc

_N_CLASSES = 128
_ROWS = 16384
_COLS = 26
_FLAT_ROWS = _ROWS * _COLS
_CHUNK = 256
_N_CHUNKS = _FLAT_ROWS // _CHUNK
_NW = 32
_BASE_T = _N_CHUNKS // _NW
_EXTRA_W = _N_CHUNKS - _BASE_T * _NW
_T_MAX = _BASE_T + 1


def _sc_body(xt_hbm, zeros_hbm, out_hbm, idxbuf, buf0, buf1, buf2,
             sem0, sem1, sem2):
    wid = lax.axis_index("s") * 2 + lax.axis_index("c")
    n_w = _BASE_T + jnp.where(wid < _EXTRA_W, 1, 0)
    start_w = _BASE_T * wid + jnp.minimum(wid, _EXTRA_W)

    pltpu.sync_copy(zeros_hbm, buf0)
    pltpu.sync_copy(zeros_hbm, buf1)
    pltpu.sync_copy(zeros_hbm, buf2)
    pltpu.sync_copy(
        xt_hbm.at[pl.ds(start_w * _CHUNK, _BASE_T * _CHUNK)],
        idxbuf.at[pl.ds(0, _BASE_T * _CHUNK)])

    @pl.when(wid < _EXTRA_W)
    def _():
        pltpu.sync_copy(
            xt_hbm.at[pl.ds((start_w + _BASE_T) * _CHUNK, _CHUNK)],
            idxbuf.at[pl.ds(_BASE_T * _CHUNK, _CHUNK)])

    lanes = lax.iota(jnp.int32, 16)
    one16 = jnp.ones((16,), jnp.int32)
    zero16 = jnp.zeros((16,), jnp.int32)
    bufs = (buf0, buf1, buf2)
    sems = (sem0, sem1, sem2)

    def do_chunk(t, buf, sem):
        cid = start_w + t

        @pl.when(t >= 3)
        def _():
            pltpu.make_async_copy(
                buf, out_hbm.at[pl.ds(cid * _CHUNK, _CHUNK)], sem).wait()
            for i in range(_CHUNK // 16):
                xv = idxbuf[pl.ds((t - 3) * _CHUNK + i * 16, 16)]
                rows = lanes + (i * 16)
                plsc.store_scatter(buf, [rows, xv], zero16)

        for i in range(_CHUNK // 16):
            xv = idxbuf[pl.ds(t * _CHUNK + i * 16, 16)]
            rows = lanes + (i * 16)
            plsc.store_scatter(buf, [rows, xv], one16)
        pltpu.async_copy(buf, out_hbm.at[pl.ds(cid * _CHUNK, _CHUNK)], sem)

    def pair_body(u, carry):
        for phase in range(3):
            t = u * 3 + phase

            @pl.when(t < n_w)
            def _():
                do_chunk(t, bufs[phase], sems[phase])
        return carry

    lax.fori_loop(0, (_T_MAX + 2) // 3, pair_body, 0)

    for b in range(3):
        pltpu.make_async_copy(
            bufs[b], out_hbm.at[pl.ds(0, _CHUNK)], sems[b]).wait()


def kernel(x):
    xt_flat = jnp.transpose(x, (1, 0)).reshape(_FLAT_ROWS)
    zeros = jnp.zeros((_CHUNK, _N_CLASSES), jnp.int32)
    mesh = plsc.VectorSubcoreMesh(
        core_axis_name="c", subcore_axis_name="s",
        num_cores=2, num_subcores=16)
    run = functools.partial(
        pl.kernel,
        out_type=jax.ShapeDtypeStruct((_FLAT_ROWS, _N_CLASSES), jnp.int32),
        mesh=mesh,
        scratch_types=[
            pltpu.VMEM((_T_MAX * _CHUNK,), jnp.int32),
            pltpu.VMEM((_CHUNK, _N_CLASSES), jnp.int32),
            pltpu.VMEM((_CHUNK, _N_CLASSES), jnp.int32),
            pltpu.VMEM((_CHUNK, _N_CLASSES), jnp.int32),
            pltpu.SemaphoreType.DMA,
            pltpu.SemaphoreType.DMA,
            pltpu.SemaphoreType.DMA,
        ],
        compiler_params=pltpu.CompilerParams(needs_layout_passes=False),
    )(_sc_body)
    out_flat = run(xt_flat, zeros)
    return jnp.transpose(
        out_flat.reshape(_COLS, _ROWS, _N_CLASSES), (1, 0, 2))


# speedup vs baseline: 1.3544x; 1.3544x over previous
import jax
import jax.numpy as jnp
from jax.experimental import pallas as pl

_N_CLASSES = 128
_ROWS = 16384
_COLS = 26
_W = _COLS * _N_CLASSES
_BLOCK = 1024


def _onehot_body(x_ref, o_ref):
    xf = x_ref[...].astype(jnp.bfloat16)
    cid = jax.lax.broadcasted_iota(jnp.int32, (_COLS, _W), 1)
    jid = jax.lax.broadcasted_iota(jnp.int32, (_COLS, _W), 0)
    expand = (cid // _N_CLASSES == jid).astype(jnp.bfloat16)
    xrep = jax.lax.dot_general(
        xf, expand,
        dimension_numbers=(((1,), (0,)), ((), ())),
        preferred_element_type=jnp.float32,
    )
    kconst = (
        jax.lax.broadcasted_iota(jnp.int32, (_BLOCK, _W), 1) % _N_CLASSES
    ).astype(jnp.float32)
    oh = (xrep == kconst).astype(jnp.int32)
    for j in range(_COLS):
        o_ref[j, :, :] = oh[:, j * _N_CLASSES:(j + 1) * _N_CLASSES]


def kernel(x):
    grid = _ROWS // _BLOCK
    out_t = pl.pallas_call(
        _onehot_body,
        grid=(grid,),
        in_specs=[pl.BlockSpec((_BLOCK, _COLS), lambda i: (i, 0))],
        out_specs=pl.BlockSpec((_COLS, _BLOCK, _N_CLASSES), lambda i: (0, i, 0)),
        out_shape=jax.ShapeDtypeStruct((_COLS, _ROWS, _N_CLASSES), jnp.int32),
    )(x)
    return jnp.transpose(out_t, (1, 0, 2))
